# baseline (speedup 1.0000x reference)
import jax
import jax.numpy as jnp
from jax.experimental import pallas as pl
from jax.experimental.pallas import tpu as pltpu

N = 4096
M = 4096
BN = 512


def _fused_kernel(x_ref, y_ref, Q_ref, AT_ref, c_ref, out_ref, acc_ref):
    i = pl.program_id(0)

    @pl.when(i == 0)
    def _init():
        acc_ref[0, 0] = 0.0
        acc_ref[0, 1] = 0.0

    c_blk = c_ref[...]
    r = (
        jnp.dot(Q_ref[...], x_ref[...], preferred_element_type=jnp.float32)
        + jnp.dot(AT_ref[...], y_ref[...], preferred_element_type=jnp.float32)
        + c_blk
    )
    acc_ref[0, 0] += jnp.sum(r * r)
    acc_ref[0, 1] += jnp.sum(c_blk * c_blk)

    @pl.when(i == pl.num_programs(0) - 1)
    def _fin():
        top = jnp.sqrt(acc_ref[0, 0])
        bot = 0.0001 + jnp.sqrt(acc_ref[0, 1])
        out_ref[...] = jnp.full((1, 1), top / bot, dtype=jnp.float32)


def kernel(Q, AT, b, c, x, y):
    del b
    c2 = c.reshape(N, 1)
    out = pl.pallas_call(
        _fused_kernel,
        grid=(N // BN,),
        in_specs=[
            pl.BlockSpec((N, 1), lambda i: (0, 0)),
            pl.BlockSpec((M, 1), lambda i: (0, 0)),
            pl.BlockSpec((BN, N), lambda i: (i, 0)),
            pl.BlockSpec((BN, M), lambda i: (i, 0)),
            pl.BlockSpec((BN, 1), lambda i: (i, 0)),
        ],
        out_specs=pl.BlockSpec((1, 1), lambda i: (0, 0)),
        out_shape=jax.ShapeDtypeStruct((1, 1), jnp.float32),
        scratch_shapes=[pltpu.SMEM((1, 2), jnp.float32)],
    )(x, y, Q, AT, c2)
    return out[0, 0]


# speedup vs baseline: 1.0591x; 1.0591x over previous
import jax
import jax.numpy as jnp
from jax.experimental import pallas as pl
from jax.experimental.pallas import tpu as pltpu

N = 4096
M = 4096
BN = 256


def _fused_kernel(x_ref, y_ref, Q_ref, AT_ref, c_ref, out_ref, acc_ref):
    i = pl.program_id(0)

    @pl.when(i == 0)
    def _init():
        acc_ref[0, 0] = 0.0
        acc_ref[0, 1] = 0.0

    c_blk = c_ref[...]
    r = (
        jnp.dot(Q_ref[...], x_ref[...], preferred_element_type=jnp.float32)
        + jnp.dot(AT_ref[...], y_ref[...], preferred_element_type=jnp.float32)
        + c_blk
    )
    acc_ref[0, 0] += jnp.sum(r * r)
    acc_ref[0, 1] += jnp.sum(c_blk * c_blk)

    @pl.when(i == pl.num_programs(0) - 1)
    def _fin():
        top = jnp.sqrt(acc_ref[0, 0])
        bot = 0.0001 + jnp.sqrt(acc_ref[0, 1])
        out_ref[...] = jnp.full((1, 1), top / bot, dtype=jnp.float32)


def kernel(Q, AT, b, c, x, y):
    del b
    c2 = c.reshape(N, 1)
    out = pl.pallas_call(
        _fused_kernel,
        grid=(N // BN,),
        in_specs=[
            pl.BlockSpec((N, 1), lambda i: (0, 0)),
            pl.BlockSpec((M, 1), lambda i: (0, 0)),
            pl.BlockSpec((BN, N), lambda i: (i, 0)),
            pl.BlockSpec((BN, M), lambda i: (i, 0)),
            pl.BlockSpec((BN, 1), lambda i: (i, 0)),
        ],
        out_specs=pl.BlockSpec((1, 1), lambda i: (0, 0)),
        out_shape=jax.ShapeDtypeStruct((1, 1), jnp.float32),
        scratch_shapes=[pltpu.SMEM((1, 2), jnp.float32)],
    )(x, y, Q, AT, c2)
    return out[0, 0]
